# Initial kernel scaffold; baseline (speedup 1.0000x reference)
import jax
import jax.numpy as jnp
from jax.experimental import pallas as pl


def kernel(input_ids, segment_ids, tok_table, pos_table, seg_table):
    raise NotImplementedError("write your pallas kernel here")


# speedup vs baseline: 5.8268x; 5.8268x over previous
import functools

import jax
import jax.numpy as jnp
from jax import lax
from jax.experimental import pallas as pl
from jax.experimental.pallas import tpu as pltpu
from jax.experimental.pallas import tpu_sc as plsc

NC, NS = 2, 16
NW = NC * NS
D = 128
LANES = 16


def _comb_body(pos_ref, seg_ref, out_ref):
    out_ref[...] = pos_ref[...][None] + seg_ref[...][:, None]


def _emb_body(T, S, tok_hbm, tidx_hbm, cidx_hbm, comb_hbm, out_hbm,
              comb_v, tidx_v, cidx_v, rows_v, sem):
    wid = lax.axis_index("s") * NC + lax.axis_index("c")
    pltpu.sync_copy(comb_hbm, comb_v)

    def chunk_body(g, carry):
        base = (wid * S + g) * T
        pltpu.sync_copy(tidx_hbm.at[pl.ds(base, T)], tidx_v)
        pltpu.sync_copy(cidx_hbm.at[pl.ds(base, T)], cidx_v)
        copies = []
        off = 0
        while off < T:
            n = min(128, T - off)
            copies.append(pltpu.async_copy(
                tok_hbm.at[tidx_v.at[pl.ds(off, n)]],
                rows_v.at[pl.ds(off, n)], sem))
            off += n
        for c in copies:
            c.wait()

        def grp_body(g2, carry):
            t0 = g2 * LANES
            cv = cidx_v[pl.ds(t0, LANES)]
            for i in range(LANES):
                s = cv[i]
                t = t0 + i
                for d in range(D // LANES):
                    sl = pl.ds(d * LANES, LANES)
                    rows_v[t, sl] = rows_v[t, sl] + comb_v[s, sl]
            return carry

        lax.fori_loop(0, T // LANES, grp_body, 0)
        pltpu.sync_copy(rows_v, out_hbm.at[pl.ds(base, T)])
        return carry

    lax.fori_loop(0, S, chunk_body, 0)


def kernel(input_ids, segment_ids, tok_table, pos_table, seg_table):
    B, L = input_ids.shape
    n_seg = seg_table.shape[0]

    comb = pl.pallas_call(
        _comb_body,
        out_shape=jax.ShapeDtypeStruct((n_seg, L, D), jnp.float32),
    )(pos_table[:L], seg_table)
    comb = comb.reshape(n_seg * L, D)

    tidx = input_ids.reshape(-1).astype(jnp.int32)
    cidx = (segment_ids.astype(jnp.int32) * L
            + jnp.arange(L, dtype=jnp.int32)[None, :]).reshape(-1)

    T = 2 * L
    S = B * L // (NW * T)
    mesh = plsc.VectorSubcoreMesh(core_axis_name="c", subcore_axis_name="s",
                                  num_cores=NC, num_subcores=NS)
    emb = pl.kernel(
        functools.partial(_emb_body, T, S),
        out_type=jax.ShapeDtypeStruct((B * L, D), jnp.float32),
        mesh=mesh,
        scratch_types=[
            pltpu.VMEM((n_seg * L, D), jnp.float32),
            pltpu.VMEM((T,), jnp.int32),
            pltpu.VMEM((T,), jnp.int32),
            pltpu.VMEM((T, D), jnp.float32),
            pltpu.SemaphoreType.DMA,
        ],
    )
    out = emb(tok_table, tidx, cidx, comb)
    return out.reshape(B, L, D)


# speedup vs baseline: 6.5405x; 1.1225x over previous
import functools

import jax
import jax.numpy as jnp
from jax import lax
from jax.experimental import pallas as pl
from jax.experimental.pallas import tpu as pltpu
from jax.experimental.pallas import tpu_sc as plsc

NC, NS = 2, 16
NW = NC * NS
D = 128
LANES = 16
NBUF = 3


def _comb_body(pos_ref, seg_ref, out_ref):
    out_ref[...] = pos_ref[...][None] + seg_ref[...][:, None]


def _emb_body(T, S, tok_hbm, pidx_hbm, comb_hbm, out_hbm,
              comb_v, idx_v, rows_v, isems, gsems, osems):
    wid = lax.axis_index("s") * NC + lax.axis_index("c")
    pltpu.sync_copy(comb_hbm, comb_v)

    def idx_start(g, b):
        pltpu.make_async_copy(pidx_hbm.at[wid * S + g], idx_v.at[b],
                              isems.at[b]).start()

    def idx_wait(g, b):
        pltpu.make_async_copy(pidx_hbm.at[wid * S + g], idx_v.at[b],
                              isems.at[b]).wait()

    def gather_start(b):
        pltpu.make_async_copy(tok_hbm.at[idx_v.at[b, 0]], rows_v.at[b],
                              gsems.at[b]).start()

    def gather_wait(b):
        pltpu.make_async_copy(tok_hbm.at[idx_v.at[b, 0]], rows_v.at[b],
                              gsems.at[b]).wait()

    def out_start(g, b):
        base = (wid * S + g) * T
        pltpu.make_async_copy(rows_v.at[b], out_hbm.at[pl.ds(base, T)],
                              osems.at[b]).start()

    def out_wait(g, b):
        base = (wid * S + g) * T
        pltpu.make_async_copy(rows_v.at[b], out_hbm.at[pl.ds(base, T)],
                              osems.at[b]).wait()

    def compute(b):
        def grp_body(g2, carry):
            t0 = g2 * LANES
            cv = idx_v[b, 1, pl.ds(t0, LANES)]
            svals = [cv[i] for i in range(LANES)]

            def d_body(d, carry2):
                sl = pl.ds(d * LANES, LANES)
                for i in range(LANES):
                    t = t0 + i
                    rows_v[b, t, sl] = rows_v[b, t, sl] + comb_v[svals[i], sl]
                return carry2

            lax.fori_loop(0, D // LANES, d_body, 0)
            return carry

        lax.fori_loop(0, T // LANES, grp_body, 0)

    def chunk_iter(g, b, bn, *, first=False, do_next=True, do_idx=True):
        if do_next:
            idx_wait(g + 1, bn)
            if not first:
                out_wait(g - 2, bn)
            gather_start(bn)
        gather_wait(b)
        compute(b)
        out_start(g, b)
        if do_idx:
            idx_start(g + 3, b)

    idx_start(0, 0)
    idx_wait(0, 0)
    gather_start(0)
    idx_start(1, 1)
    idx_start(2, 2)
    chunk_iter(0, 0, 1, first=True)
    chunk_iter(1, 1, 2, first=True)

    def mid_body(i, carry):
        g = 3 * i + 2
        chunk_iter(g, 2, 0)
        chunk_iter(g + 1, 0, 1)
        chunk_iter(g + 2, 1, 2)
        return carry

    lax.fori_loop(0, (S - 5) // 3, mid_body, 0)

    chunk_iter(S - 3, 2, 0, do_idx=False)
    chunk_iter(S - 2, 0, 1, do_idx=False)
    chunk_iter(S - 1, 1, 2, do_next=False, do_idx=False)
    out_wait(S - 3, 2)
    out_wait(S - 2, 0)
    out_wait(S - 1, 1)


def kernel(input_ids, segment_ids, tok_table, pos_table, seg_table):
    B, L = input_ids.shape
    n_seg = seg_table.shape[0]

    comb = pl.pallas_call(
        _comb_body,
        out_shape=jax.ShapeDtypeStruct((n_seg, L, D), jnp.float32),
    )(pos_table[:L], seg_table)
    comb = comb.reshape(n_seg * L, D)

    tidx = input_ids.reshape(-1).astype(jnp.int32)
    cidx = (segment_ids.astype(jnp.int32) * L
            + jnp.arange(L, dtype=jnp.int32)[None, :]).reshape(-1)

    T = 128
    n_chunks = B * L // T
    S = n_chunks // NW
    assert (S - 5) % 3 == 0
    pidx = jnp.stack([tidx.reshape(n_chunks, T), cidx.reshape(n_chunks, T)],
                     axis=1)

    mesh = plsc.VectorSubcoreMesh(core_axis_name="c", subcore_axis_name="s",
                                  num_cores=NC, num_subcores=NS)
    emb = pl.kernel(
        functools.partial(_emb_body, T, S),
        out_type=jax.ShapeDtypeStruct((B * L, D), jnp.float32),
        mesh=mesh,
        scratch_types=[
            pltpu.VMEM((n_seg * L, D), jnp.float32),
            pltpu.VMEM((NBUF, 2, T), jnp.int32),
            pltpu.VMEM((NBUF, T, D), jnp.float32),
            pltpu.SemaphoreType.DMA((NBUF,)),
            pltpu.SemaphoreType.DMA((NBUF,)),
            pltpu.SemaphoreType.DMA((NBUF,)),
        ],
    )
    out = emb(tok_table, pidx, comb)
    return out.reshape(B, L, D)


# speedup vs baseline: 9.7561x; 1.4916x over previous
import functools

import jax
import jax.numpy as jnp
from jax import lax
from jax.experimental import pallas as pl
from jax.experimental.pallas import tpu as pltpu
from jax.experimental.pallas import tpu_sc as plsc

NC, NS = 2, 16
NW = NC * NS
D = 128
LANES = 16
NBUF = 3
KU = 8


def _comb_body(pos_ref, seg_ref, out_ref):
    out_ref[...] = pos_ref[...][None] + seg_ref[...][:, None]


def _emb_body(T, S, tok_hbm, pidx_hbm, comb_hbm, out_hbm,
              idx_v, rows_v, isems, gsems, asems, osems):
    wid = lax.axis_index("s") * NC + lax.axis_index("c")

    def idx_start(g, b):
        pltpu.make_async_copy(pidx_hbm.at[wid * S + g], idx_v.at[b],
                              isems.at[b]).start()

    def idx_wait(g, b):
        pltpu.make_async_copy(pidx_hbm.at[wid * S + g], idx_v.at[b],
                              isems.at[b]).wait()

    def gather_start(b):
        pltpu.make_async_copy(tok_hbm.at[idx_v.at[b, 0]], rows_v.at[b],
                              gsems.at[b]).start()

    def gather_wait(b):
        pltpu.make_async_copy(tok_hbm.at[idx_v.at[b, 0]], rows_v.at[b],
                              gsems.at[b]).wait()

    def addg(b):
        pltpu.async_copy(comb_hbm.at[idx_v.at[b, 1]], rows_v.at[b],
                         asems.at[b], add=True).wait()

    def out_start(g, b):
        base = (wid * S + g) * T
        pltpu.make_async_copy(rows_v.at[b], out_hbm.at[pl.ds(base, T)],
                              osems.at[b]).start()

    def out_wait(g, b):
        base = (wid * S + g) * T
        pltpu.make_async_copy(rows_v.at[b], out_hbm.at[pl.ds(base, T)],
                              osems.at[b]).wait()

    def chunk_iter(g, b, bn, *, first=False, do_next=True, do_idx=True):
        if do_next:
            idx_wait(g + 1, bn)
            if not first:
                out_wait(g - 2, bn)
            gather_start(bn)
        gather_wait(b)
        addg(b)
        out_start(g, b)
        if do_idx:
            idx_start(g + 3, b)

    idx_start(0, 0)
    idx_wait(0, 0)
    gather_start(0)
    idx_start(1, 1)
    idx_start(2, 2)
    chunk_iter(0, 0, 1, first=True)
    chunk_iter(1, 1, 2, first=True)

    def mid_body(i, carry):
        g = 3 * i + 2
        chunk_iter(g, 2, 0)
        chunk_iter(g + 1, 0, 1)
        chunk_iter(g + 2, 1, 2)
        return carry

    lax.fori_loop(0, (S - 5) // 3, mid_body, 0)

    chunk_iter(S - 3, 2, 0, do_idx=False)
    chunk_iter(S - 2, 0, 1, do_idx=False)
    chunk_iter(S - 1, 1, 2, do_next=False, do_idx=False)
    out_wait(S - 3, 2)
    out_wait(S - 2, 0)
    out_wait(S - 1, 1)


def kernel(input_ids, segment_ids, tok_table, pos_table, seg_table):
    B, L = input_ids.shape
    n_seg = seg_table.shape[0]

    comb = pl.pallas_call(
        _comb_body,
        out_shape=jax.ShapeDtypeStruct((n_seg, L, D), jnp.float32),
    )(pos_table[:L], seg_table)
    comb = comb.reshape(n_seg * L, D)

    tidx = input_ids.reshape(-1).astype(jnp.int32)
    cidx = (segment_ids.astype(jnp.int32) * L
            + jnp.arange(L, dtype=jnp.int32)[None, :]).reshape(-1)

    T = 128
    n_chunks = B * L // T
    S = n_chunks // NW
    assert (S - 5) % 3 == 0
    pidx = jnp.stack([tidx.reshape(n_chunks, T), cidx.reshape(n_chunks, T)],
                     axis=1)

    mesh = plsc.VectorSubcoreMesh(core_axis_name="c", subcore_axis_name="s",
                                  num_cores=NC, num_subcores=NS)
    emb = pl.kernel(
        functools.partial(_emb_body, T, S),
        out_type=jax.ShapeDtypeStruct((B * L, D), jnp.float32),
        mesh=mesh,
        scratch_types=[
            pltpu.VMEM((NBUF, 2, T), jnp.int32),
            pltpu.VMEM((NBUF, T, D), jnp.float32),
            pltpu.SemaphoreType.DMA((NBUF,)),
            pltpu.SemaphoreType.DMA((NBUF,)),
            pltpu.SemaphoreType.DMA((NBUF,)),
            pltpu.SemaphoreType.DMA((NBUF,)),
        ],
    )
    out = emb(tok_table, pidx, comb)
    return out.reshape(B, L, D)


# speedup vs baseline: 20.4588x; 2.0970x over previous
import functools

import jax
import jax.numpy as jnp
from jax import lax
from jax.experimental import pallas as pl
from jax.experimental.pallas import tpu as pltpu
from jax.experimental.pallas import tpu_sc as plsc

NC, NS = 2, 16
NW = NC * NS
D = 128
LANES = 16
NBUF = 3
KU = 8


def _comb_body(pos_ref, seg_ref, out_ref):
    out_ref[...] = pos_ref[...][None] + seg_ref[...][:, None]


def _emb_body(T, S, tok_hbm, pidx_hbm, comb_hbm, out_hbm,
              comb_sh, idx_v, rows_v, isems, gsems, asems, osems):
    wid = lax.axis_index("s") * NC + lax.axis_index("c")

    @pl.when(lax.axis_index("s") == 0)
    def _():
        pltpu.sync_copy(comb_hbm, comb_sh)

    plsc.subcore_barrier()

    def idx_start(g, b):
        pltpu.make_async_copy(pidx_hbm.at[wid * S + g], idx_v.at[b],
                              isems.at[b]).start()

    def idx_wait(g, b):
        pltpu.make_async_copy(pidx_hbm.at[wid * S + g], idx_v.at[b],
                              isems.at[b]).wait()

    def gather_start(b):
        pltpu.make_async_copy(tok_hbm.at[idx_v.at[b, 0]], rows_v.at[b],
                              gsems.at[b]).start()

    def gather_wait(b):
        pltpu.make_async_copy(tok_hbm.at[idx_v.at[b, 0]], rows_v.at[b],
                              gsems.at[b]).wait()

    def addg(b):
        pltpu.async_copy(comb_sh.at[idx_v.at[b, 1]], rows_v.at[b],
                         asems.at[b], add=True).wait()

    def out_start(g, b):
        base = (wid * S + g) * T
        pltpu.make_async_copy(rows_v.at[b], out_hbm.at[pl.ds(base, T)],
                              osems.at[b]).start()

    def out_wait(g, b):
        base = (wid * S + g) * T
        pltpu.make_async_copy(rows_v.at[b], out_hbm.at[pl.ds(base, T)],
                              osems.at[b]).wait()

    def chunk_iter(g, b, bn, *, first=False, do_next=True, do_idx=True):
        if do_next:
            idx_wait(g + 1, bn)
            if not first:
                out_wait(g - 2, bn)
            gather_start(bn)
        gather_wait(b)
        addg(b)
        out_start(g, b)
        if do_idx:
            idx_start(g + 3, b)

    idx_start(0, 0)
    idx_wait(0, 0)
    gather_start(0)
    idx_start(1, 1)
    idx_start(2, 2)
    chunk_iter(0, 0, 1, first=True)
    chunk_iter(1, 1, 2, first=True)

    def mid_body(i, carry):
        g = 3 * i + 2
        chunk_iter(g, 2, 0)
        chunk_iter(g + 1, 0, 1)
        chunk_iter(g + 2, 1, 2)
        return carry

    lax.fori_loop(0, (S - 5) // 3, mid_body, 0)

    chunk_iter(S - 3, 2, 0, do_idx=False)
    chunk_iter(S - 2, 0, 1, do_idx=False)
    chunk_iter(S - 1, 1, 2, do_next=False, do_idx=False)
    out_wait(S - 3, 2)
    out_wait(S - 2, 0)
    out_wait(S - 1, 1)


def kernel(input_ids, segment_ids, tok_table, pos_table, seg_table):
    B, L = input_ids.shape
    n_seg = seg_table.shape[0]

    comb = pl.pallas_call(
        _comb_body,
        out_shape=jax.ShapeDtypeStruct((n_seg, L, D), jnp.float32),
    )(pos_table[:L], seg_table)
    comb = comb.reshape(n_seg * L, D)

    tidx = input_ids.reshape(-1).astype(jnp.int32)
    cidx = (segment_ids.astype(jnp.int32) * L
            + jnp.arange(L, dtype=jnp.int32)[None, :]).reshape(-1)

    T = 128
    n_chunks = B * L // T
    S = n_chunks // NW
    assert (S - 5) % 3 == 0
    pidx = jnp.stack([tidx.reshape(n_chunks, T), cidx.reshape(n_chunks, T)],
                     axis=1)

    mesh = plsc.VectorSubcoreMesh(core_axis_name="c", subcore_axis_name="s",
                                  num_cores=NC, num_subcores=NS)
    emb = pl.kernel(
        functools.partial(_emb_body, T, S),
        out_type=jax.ShapeDtypeStruct((B * L, D), jnp.float32),
        mesh=mesh,
        scratch_types=[
            pltpu.VMEM_SHARED((n_seg * L, D), jnp.float32),
            pltpu.VMEM((NBUF, 2, T), jnp.int32),
            pltpu.VMEM((NBUF, T, D), jnp.float32),
            pltpu.SemaphoreType.DMA((NBUF,)),
            pltpu.SemaphoreType.DMA((NBUF,)),
            pltpu.SemaphoreType.DMA((NBUF,)),
            pltpu.SemaphoreType.DMA((NBUF,)),
        ],
    )
    out = emb(tok_table, pidx, comb)
    return out.reshape(B, L, D)
